# baseline (speedup 1.0000x reference)
import jax
import jax.numpy as jnp
from jax import lax
from jax.experimental import pallas as pl
from jax.experimental.pallas import tpu as pltpu
from jax.experimental.pallas import tpu_sc as plsc

MAXLEN = 2048
EMBED_DIM = 2048
BATCH = 4

NUM_CORES = 2
NUM_SUBCORES = 16
NUM_WORKERS = NUM_CORES * NUM_SUBCORES
T_PER_WORKER = MAXLEN // NUM_WORKERS
T_CHUNK = 8
N_CHUNKS = T_PER_WORKER // T_CHUNK
N_STEPS = N_CHUNKS * BATCH
CHUNK_ELEMS = T_CHUNK * EMBED_DIM
LANES = 16
ROW_SHIFT = 11


NBUF = 4
PREFETCH = 2


def _sc_body(x_hbm, pos_hbm, out_hbm,
             posbuf0, posbuf1, xbuf0, xbuf1, xbuf2, xbuf3,
             sem_p0, sem_p1, sem_x0, sem_x1, sem_x2, sem_x3,
             sem_s0, sem_s1, sem_s2, sem_s3):
    wid = lax.axis_index("s") * NUM_CORES + lax.axis_index("c")
    t_base = wid * T_PER_WORKER
    posbufs = (posbuf0, posbuf1)
    xbufs = (xbuf0, xbuf1, xbuf2, xbuf3)
    sems_p = (sem_p0, sem_p1)
    sems_x = (sem_x0, sem_x1, sem_x2, sem_x3)
    sems_s = (sem_s0, sem_s1, sem_s2, sem_s3)

    def pos_load(c):
        return pltpu.async_copy(
            pos_hbm.at[pl.ds(t_base + c * T_CHUNK, T_CHUNK)],
            posbufs[c % 2], sems_p[c % 2])

    def x_row(k):
        c, b = divmod(k, BATCH)
        return b * MAXLEN + t_base + c * T_CHUNK

    def x_load(k):
        return pltpu.async_copy(
            x_hbm.at[pl.ds(x_row(k), T_CHUNK)], xbufs[k % NBUF],
            sems_x[k % NBUF])

    def out_store(k):
        return pltpu.async_copy(
            xbufs[k % NBUF], out_hbm.at[pl.ds(x_row(k), T_CHUNK)],
            sems_s[k % NBUF])

    pos_handles = [pos_load(0)]
    x_handles = [x_load(k) for k in range(PREFETCH + 1)]
    store_handles = []

    for k in range(N_STEPS):
        c, b = divmod(k, BATCH)
        if b == 0:
            if c + 1 < N_CHUNKS:
                pos_handles.append(pos_load(c + 1))
            pos_handles.pop(0).wait()
        nxt = k + 1 + PREFETCH
        if nxt < N_STEPS:
            prev_store = nxt - NBUF
            if prev_store >= 0:
                store_handles[prev_store].wait()
            x_handles.append(x_load(nxt))
        x_handles.pop(0).wait()

        pbuf = posbufs[c % 2]
        xbuf = xbufs[k % NBUF]

        @plsc.parallel_loop(0, CHUNK_ELEMS, LANES, unroll=16)
        def _(i):
            r = lax.shift_right_logical(i, ROW_SHIFT)
            col = pl.multiple_of(lax.bitwise_and(i, EMBED_DIM - 1), LANES)
            plsc.addupdate(xbuf.at[r, pl.ds(col, LANES)],
                           pbuf[r, pl.ds(col, LANES)])

        store_handles.append(out_store(k))

    for k in range(N_STEPS - NBUF, N_STEPS):
        store_handles[k].wait()


def kernel(x, pos_table):
    mesh = plsc.VectorSubcoreMesh(core_axis_name="c", subcore_axis_name="s")
    run = pl.kernel(
        _sc_body,
        mesh=mesh,
        out_type=jax.ShapeDtypeStruct((BATCH * MAXLEN, EMBED_DIM), jnp.float32),
        scratch_types=(
            [pltpu.VMEM((T_CHUNK, EMBED_DIM), jnp.float32)] * (2 + NBUF)
            + [pltpu.SemaphoreType.DMA] * (2 + 2 * NBUF)
        ),
    )
    out = run(x.reshape(BATCH * MAXLEN, EMBED_DIM), pos_table)
    return out.reshape(BATCH, MAXLEN, EMBED_DIM)


# speedup vs baseline: 1.0973x; 1.0973x over previous
import jax
import jax.numpy as jnp
from jax import lax
from jax.experimental import pallas as pl
from jax.experimental.pallas import tpu as pltpu
from jax.experimental.pallas import tpu_sc as plsc

MAXLEN = 2048
EMBED_DIM = 2048
BATCH = 4

NUM_CORES = 2
NUM_SUBCORES = 16
NUM_WORKERS = NUM_CORES * NUM_SUBCORES
T_PER_WORKER = MAXLEN // NUM_WORKERS
T_CHUNK = 4
N_CHUNKS = T_PER_WORKER // T_CHUNK
N_STEPS = N_CHUNKS * BATCH
CHUNK_ELEMS = T_CHUNK * EMBED_DIM
LANES = 16
ROW_SHIFT = 11


NBUF = 8
PREFETCH = 4


def _sc_body(x_hbm, pos_hbm, out_hbm, *scratch):
    wid = lax.axis_index("s") * NUM_CORES + lax.axis_index("c")
    t_base = wid * T_PER_WORKER
    posbufs = scratch[0:2]
    xbufs = scratch[2:2 + NBUF]
    sems_p = scratch[2 + NBUF:4 + NBUF]
    sems_x = scratch[4 + NBUF:4 + 2 * NBUF]
    sems_s = scratch[4 + 2 * NBUF:4 + 3 * NBUF]

    def pos_load(c):
        return pltpu.async_copy(
            pos_hbm.at[pl.ds(t_base + c * T_CHUNK, T_CHUNK)],
            posbufs[c % 2], sems_p[c % 2])

    def x_row(k):
        c, b = divmod(k, BATCH)
        return b * MAXLEN + t_base + c * T_CHUNK

    def x_load(k):
        return pltpu.async_copy(
            x_hbm.at[pl.ds(x_row(k), T_CHUNK)], xbufs[k % NBUF],
            sems_x[k % NBUF])

    def out_store(k):
        return pltpu.async_copy(
            xbufs[k % NBUF], out_hbm.at[pl.ds(x_row(k), T_CHUNK)],
            sems_s[k % NBUF])

    pos_handles = [pos_load(0)]
    x_handles = [x_load(k) for k in range(PREFETCH + 1)]
    store_handles = []

    for k in range(N_STEPS):
        c, b = divmod(k, BATCH)
        if b == 0:
            if c + 1 < N_CHUNKS:
                pos_handles.append(pos_load(c + 1))
            pos_handles.pop(0).wait()
        nxt = k + 1 + PREFETCH
        if nxt < N_STEPS:
            prev_store = nxt - NBUF
            if prev_store >= 0:
                store_handles[prev_store].wait()
            x_handles.append(x_load(nxt))
        x_handles.pop(0).wait()

        pbuf = posbufs[c % 2]
        xbuf = xbufs[k % NBUF]

        @plsc.parallel_loop(0, CHUNK_ELEMS, LANES, unroll=8)
        def _(i):
            r = lax.shift_right_logical(i, ROW_SHIFT)
            col = pl.multiple_of(lax.bitwise_and(i, EMBED_DIM - 1), LANES)
            plsc.addupdate(xbuf.at[r, pl.ds(col, LANES)],
                           pbuf[r, pl.ds(col, LANES)])

        store_handles.append(out_store(k))

    for k in range(N_STEPS - NBUF, N_STEPS):
        store_handles[k].wait()


def kernel(x, pos_table):
    mesh = plsc.VectorSubcoreMesh(core_axis_name="c", subcore_axis_name="s")
    run = pl.kernel(
        _sc_body,
        mesh=mesh,
        out_type=jax.ShapeDtypeStruct((BATCH * MAXLEN, EMBED_DIM), jnp.float32),
        scratch_types=(
            [pltpu.VMEM((T_CHUNK, EMBED_DIM), jnp.float32)] * (2 + NBUF)
            + [pltpu.SemaphoreType.DMA] * (2 + 2 * NBUF)
        ),
    )
    out = run(x.reshape(BATCH * MAXLEN, EMBED_DIM), pos_table)
    return out.reshape(BATCH, MAXLEN, EMBED_DIM)
